# baseline (speedup 1.0000x reference)
import functools

import jax
import jax.numpy as jnp
from jax import lax
from jax.experimental import pallas as pl
from jax.experimental.pallas import tpu as pltpu
from jax.experimental.pallas import tpu_sc as plsc

NC, NS, L = 2, 16, 16
NW = NC * NS
CH = 8
NB = 4


def kernel(x, voxel_level, positional_encoding_table):
    b, s, d = x.shape
    n = b * s
    xf = x.reshape(n, d)
    idx = voxel_level.astype(jnp.int32).reshape(n)
    b_per_w = n // NW
    n_ch = b_per_w // CH
    mesh = plsc.VectorSubcoreMesh(core_axis_name="c", subcore_axis_name="s")

    buf = lambda: pltpu.VMEM((CH, d), x.dtype)
    scratch = [pltpu.VMEM((b_per_w,), jnp.int32)]
    scratch += [buf() for _ in range(3 * NB)]
    scratch += [pltpu.SemaphoreType.DMA for _ in range(3 * NB)]

    @functools.partial(
        pl.kernel, mesh=mesh,
        out_type=jax.ShapeDtypeStruct((n, d), x.dtype),
        scratch_types=scratch,
    )
    def sc_kern(idx_hbm, x_hbm, t_hbm, o_hbm, idx_v, *rest):
        pe_v = rest[0:NB]
        x_v = rest[NB:2 * NB]
        ob_v = rest[2 * NB:3 * NB]
        sem_g = rest[3 * NB:4 * NB]
        sem_x = rest[4 * NB:5 * NB]
        sem_o = rest[5 * NB:6 * NB]

        wid = lax.axis_index("s") * NC + lax.axis_index("c")
        base = wid * b_per_w
        pltpu.sync_copy(idx_hbm.at[pl.ds(base, b_per_w)], idx_v)

        def issue(c, bslot):
            pltpu.async_copy(
                t_hbm.at[idx_v.at[pl.ds(c * CH, CH)]], pe_v[bslot],
                sem_g[bslot])
            pltpu.async_copy(
                x_hbm.at[pl.ds(base + c * CH, CH)], x_v[bslot],
                sem_x[bslot])

        for bslot in range(NB):
            issue(bslot, bslot)

        @pl.loop(0, n_ch, step=NB)
        def _group(ci):
            for bslot in range(NB):
                c = ci + bslot
                pltpu.make_async_copy(
                    t_hbm.at[idx_v.at[pl.ds(c * CH, CH)]], pe_v[bslot],
                    sem_g[bslot]).wait()
                pltpu.make_async_copy(
                    x_hbm.at[pl.ds(base + c * CH, CH)], x_v[bslot],
                    sem_x[bslot]).wait()

                @pl.when(ci > 0)
                def _drain():
                    pltpu.make_async_copy(
                        ob_v[bslot],
                        o_hbm.at[pl.ds(base + (c - NB) * CH, CH)],
                        sem_o[bslot]).wait()

                @pl.loop(0, CH)
                def _row(r):
                    @pl.loop(0, d, step=L)
                    def _col(cc):
                        slc = (pl.ds(r, 1), pl.ds(cc, L))
                        ob_v[bslot].at[*slc][...] = (
                            x_v[bslot].at[*slc][...]
                            + pe_v[bslot].at[*slc][...]
                        )

                pltpu.async_copy(
                    ob_v[bslot], o_hbm.at[pl.ds(base + c * CH, CH)],
                    sem_o[bslot])

                @pl.when(c + NB < n_ch)
                def _refill():
                    issue(c + NB, bslot)

        for bslot in range(NB):
            c = n_ch - NB + bslot
            pltpu.make_async_copy(
                ob_v[bslot], o_hbm.at[pl.ds(base + c * CH, CH)],
                sem_o[bslot]).wait()

    return sc_kern(idx, xf, positional_encoding_table).reshape(b, s, d)


# speedup vs baseline: 1.4725x; 1.4725x over previous
import functools

import jax
import jax.numpy as jnp
from jax import lax
from jax.experimental import pallas as pl
from jax.experimental.pallas import tpu as pltpu
from jax.experimental.pallas import tpu_sc as plsc

NC, NS, L = 2, 16, 16
NW = NC * NS
CH = 8
NB = 4


def kernel(x, voxel_level, positional_encoding_table):
    b, s, d = x.shape
    n = b * s
    xf = x.reshape(n, d)
    idx = voxel_level.astype(jnp.int32).reshape(n)
    b_per_w = n // NW
    n_ch = b_per_w // CH
    mesh = plsc.VectorSubcoreMesh(core_axis_name="c", subcore_axis_name="s")

    buf = lambda: pltpu.VMEM((CH, d), x.dtype)
    scratch = [pltpu.VMEM((b_per_w,), jnp.int32)]
    scratch += [buf() for _ in range(3 * NB)]
    scratch += [pltpu.SemaphoreType.DMA for _ in range(3 * NB)]

    @functools.partial(
        pl.kernel, mesh=mesh,
        out_type=jax.ShapeDtypeStruct((n, d), x.dtype),
        scratch_types=scratch,
    )
    def sc_kern(idx_hbm, x_hbm, t_hbm, o_hbm, idx_v, *rest):
        pe_v = rest[0:NB]
        x_v = rest[NB:2 * NB]
        ob_v = rest[2 * NB:3 * NB]
        sem_g = rest[3 * NB:4 * NB]
        sem_x = rest[4 * NB:5 * NB]
        sem_o = rest[5 * NB:6 * NB]

        wid = lax.axis_index("s") * NC + lax.axis_index("c")
        base = wid * b_per_w
        pltpu.sync_copy(idx_hbm.at[pl.ds(base, b_per_w)], idx_v)

        def issue(c, bslot):
            pltpu.async_copy(
                t_hbm.at[idx_v.at[pl.ds(c * CH, CH)]], pe_v[bslot],
                sem_g[bslot])
            pltpu.async_copy(
                x_hbm.at[pl.ds(base + c * CH, CH)], x_v[bslot],
                sem_x[bslot])

        for bslot in range(NB):
            issue(bslot, bslot)

        @pl.loop(0, n_ch, step=NB)
        def _group(ci):
            for bslot in range(NB):
                c = ci + bslot
                pltpu.make_async_copy(
                    t_hbm.at[idx_v.at[pl.ds(c * CH, CH)]], pe_v[bslot],
                    sem_g[bslot]).wait()
                pltpu.make_async_copy(
                    x_hbm.at[pl.ds(base + c * CH, CH)], x_v[bslot],
                    sem_x[bslot]).wait()

                @pl.when(ci > 0)
                def _drain():
                    pltpu.make_async_copy(
                        ob_v[bslot],
                        o_hbm.at[pl.ds(base + (c - NB) * CH, CH)],
                        sem_o[bslot]).wait()

                @pl.loop(0, CH)
                def _row(r):
                    @plsc.parallel_loop(0, d, step=L, unroll=8)
                    def _col(cc):
                        slc = (pl.ds(r, 1), pl.ds(cc, L))
                        ob_v[bslot].at[*slc][...] = (
                            x_v[bslot].at[*slc][...]
                            + pe_v[bslot].at[*slc][...]
                        )

                pltpu.async_copy(
                    ob_v[bslot], o_hbm.at[pl.ds(base + c * CH, CH)],
                    sem_o[bslot])

                @pl.when(c + NB < n_ch)
                def _refill():
                    issue(c + NB, bslot)

        for bslot in range(NB):
            c = n_ch - NB + bslot
            pltpu.make_async_copy(
                ob_v[bslot], o_hbm.at[pl.ds(base + c * CH, CH)],
                sem_o[bslot]).wait()

    return sc_kern(idx, xf, positional_encoding_table).reshape(b, s, d)
